# baseline (speedup 1.0000x reference)
import jax
import jax.numpy as jnp
from jax.experimental import pallas as pl
from jax.experimental.pallas import tpu as pltpu

_B, _D, _N = 16, 3, 65536
_K = 1024


def _norm_body(p1_ref, pc_ref, out_ref):
    b = pl.program_id(0)
    d = pc_ref[0]
    qx = p1_ref[b, 0]
    qy = p1_ref[b, 1]
    qz = p1_ref[b, 2]
    dx = d[0:1, :] - qx
    dy = d[1:2, :] - qy
    dz = d[2:3, :] - qz
    s = (dx * dx + dy * dy) + dz * dz
    out_ref[0] = jnp.sqrt(s)


def _norms(pcloud, P1):
    return pl.pallas_call(
        _norm_body,
        grid=(_B,),
        in_specs=[
            pl.BlockSpec((_B, _D), lambda b: (0, 0), memory_space=pltpu.SMEM),
            pl.BlockSpec((1, _D, _N), lambda b: (b, 0, 0)),
        ],
        out_specs=pl.BlockSpec((1, 1, _N), lambda b: (b, 0, 0)),
        out_shape=jax.ShapeDtypeStruct((_B, 1, _N), jnp.float32),
    )(P1, pcloud)


def kernel(pcloud, P1, K):
    dist = _norms(pcloud, P1)[:, 0, :]
    _, indices = jax.lax.top_k(-dist, _K)
    indices = indices + (jnp.asarray(K) - _K).astype(indices.dtype)
    pc = jnp.transpose(pcloud, (0, 2, 1))
    a = jnp.take_along_axis(pc, indices[:, :, None], axis=1)
    a = jnp.transpose(a, (0, 2, 1))
    b = indices.astype(jnp.int32)
    return (a, b)


# speedup vs baseline: 3.1997x; 3.1997x over previous
import functools

import jax
import jax.numpy as jnp
from jax import lax
from jax.experimental import pallas as pl
from jax.experimental.pallas import tpu as pltpu
from jax.experimental.pallas import tpu_sc as plsc

_B, _D, _N = 16, 3, 65536
_K = 1024
_L = 16
_BINS = 2048
_SHIFT = 20
_NV = _N // _L
_CAND = 2048
_PADI = 0x7FFFFFFF


def _norm_body(p1_ref, pc_ref, out_ref):
    b = pl.program_id(0)
    d = pc_ref[0]
    qx = p1_ref[b, 0]
    qy = p1_ref[b, 1]
    qz = p1_ref[b, 2]
    dx = d[0:1, :] - qx
    dy = d[1:2, :] - qy
    dz = d[2:3, :] - qz
    s = (dx * dx + dy * dy) + dz * dz
    out_ref[0] = jnp.sqrt(s)


def _norms(pcloud, P1):
    return pl.pallas_call(
        _norm_body,
        grid=(_B,),
        in_specs=[
            pl.BlockSpec((_B, _D), lambda b: (0, 0), memory_space=pltpu.SMEM),
            pl.BlockSpec((1, _D, _N), lambda b: (b, 0, 0)),
        ],
        out_specs=pl.BlockSpec((1, 1, _N), lambda b: (b, 0, 0)),
        out_shape=jax.ShapeDtypeStruct((_B, 1, _N), jnp.float32),
    )(P1, pcloud)


def _scalar(x):
    return jnp.max(x) if getattr(x, "ndim", 0) else x


def _permute(x, perm):
    dn = lax.GatherDimensionNumbers(
        offset_dims=(), collapsed_slice_dims=(0,), start_index_map=(0,))
    return lax.gather(x, perm[:, None], dimension_numbers=dn, slice_sizes=(1,),
                      mode=lax.GatherScatterMode.PROMISE_IN_BOUNDS)


def _sc_body(dist, pc, a_out, b_out,
             keys_v, hist_v, skey_v, sidx_v, bkey_v, bidx_v, aout_v, gidx_v,
             sem):
    wid = lax.axis_index("s") * 2 + lax.axis_index("c")

    @pl.when(wid < _B)
    def _():
        b = wid
        iota = lax.iota(jnp.int32, _L)
        ones = jnp.ones((_L,), jnp.int32)
        zeros = jnp.zeros((_L,), jnp.int32)
        inf16 = jnp.full((_L,), jnp.inf, jnp.float32)
        padi16 = jnp.full((_L,), _PADI, jnp.int32)

        pltpu.sync_copy(dist.at[pl.ds(b * _N, _N)], keys_v)

        def zero_body(i, _):
            hist_v[pl.ds(i * _L, _L)] = zeros
            return 0
        lax.fori_loop(0, _BINS, zero_body, 0)

        def hist_body(i, _):
            kv = keys_v[pl.ds(i * _L, _L)]
            bits = lax.bitcast_convert_type(kv, jnp.int32)
            binv = lax.shift_right_logical(bits, _SHIFT)
            addr = iota * _BINS + binv
            plsc.addupdate_scatter(hist_v, [addr], ones)
            return 0
        lax.fori_loop(0, _NV, hist_body, 0)

        def red_body(j, _):
            acc = hist_v[pl.ds(j * _L, _L)]
            for r in range(1, _L):
                acc = acc + hist_v[pl.ds(r * _BINS + j * _L, _L)]
            hist_v[pl.ds(j * _L, _L)] = acc
            return 0
        lax.fori_loop(0, _BINS // _L, red_body, 0)

        def scan_body(j, carry):
            total, bstar, found = carry
            h16 = hist_v[pl.ds(j * _L, _L)]
            c16 = plsc.cumsum(h16)
            chunk = jnp.max(c16)
            cum = total + c16
            cross = cum >= _K
            crossed = jnp.logical_and(total + chunk >= _K, found == 0)
            pos = _scalar(plsc.all_reduce_ffs(cross))
            bstar = jnp.where(crossed, j * _L + pos, bstar)
            found = jnp.where(crossed, 1, found)
            return (total + chunk, bstar, found)
        _, bstar, _ = lax.fori_loop(
            0, _BINS // _L, scan_body,
            (jnp.int32(0), jnp.int32(0), jnp.int32(0)))

        def pad_body(i, _):
            skey_v[pl.ds(i * _L, _L)] = inf16
            sidx_v[pl.ds(i * _L, _L)] = padi16
            bkey_v[pl.ds(i * _L, _L)] = inf16
            bidx_v[pl.ds(i * _L, _L)] = padi16
            return 0
        lax.fori_loop(0, (_CAND + _L) // _L, pad_body, 0)

        def comp_body(i, carry):
            nd, nb = carry
            kv = keys_v[pl.ds(i * _L, _L)]
            bits = lax.bitcast_convert_type(kv, jnp.int32)
            binv = lax.shift_right_logical(bits, _SHIFT)
            idx16 = i * _L + iota
            mdef = binv < bstar
            mbor = binv == bstar
            plsc.store_compressed(skey_v.at[pl.ds(nd, _L)], kv, mask=mdef)
            plsc.store_compressed(sidx_v.at[pl.ds(nd, _L)], idx16, mask=mdef)
            plsc.store_compressed(bkey_v.at[pl.ds(nb, _L)], kv, mask=mbor)
            plsc.store_compressed(bidx_v.at[pl.ds(nb, _L)], idx16, mask=mbor)
            nd = nd + _scalar(plsc.all_reduce_population_count(mdef))
            nb = nb + _scalar(plsc.all_reduce_population_count(mbor))
            nb = jnp.minimum(nb, jnp.int32(_CAND - _L))
            return (nd, nb)
        nd, nb = lax.fori_loop(0, _NV, comp_body,
                               (jnp.int32(0), jnp.int32(0)))

        nb_c = jnp.minimum(nb, jnp.int32(_CAND) - nd)

        def app_body(i, _):
            skey_v[pl.ds(nd + i * _L, _L)] = bkey_v[pl.ds(i * _L, _L)]
            sidx_v[pl.ds(nd + i * _L, _L)] = bidx_v[pl.ds(i * _L, _L)]
            return 0
        lax.fori_loop(0, (nb_c + _L - 1) // _L, app_body, 0)

        nvec = _CAND // _L
        k = 2
        while k <= _CAND:
            j = k // 2
            while j >= 1:
                if j >= _L:
                    dd = j // _L
                    s = dd.bit_length() - 1

                    def inter_body(u, _, k=k, dd=dd, s=s):
                        v_lo = ((u >> s) << (s + 1)) | (u & (dd - 1))
                        v_hi = v_lo + dd
                        ak = skey_v[pl.ds(v_lo * _L, _L)]
                        ai = sidx_v[pl.ds(v_lo * _L, _L)]
                        bk = skey_v[pl.ds(v_hi * _L, _L)]
                        bi = sidx_v[pl.ds(v_hi * _L, _L)]
                        asc = ((v_lo * _L) & k) == 0
                        altb = jnp.logical_or(
                            ak < bk, jnp.logical_and(ak == bk, ai < bi))
                        sel = altb == jnp.broadcast_to(asc, (_L,))
                        skey_v[pl.ds(v_lo * _L, _L)] = jnp.where(sel, ak, bk)
                        sidx_v[pl.ds(v_lo * _L, _L)] = jnp.where(sel, ai, bi)
                        skey_v[pl.ds(v_hi * _L, _L)] = jnp.where(sel, bk, ak)
                        sidx_v[pl.ds(v_hi * _L, _L)] = jnp.where(sel, bi, ai)
                        return 0
                    lax.fori_loop(0, nvec // 2, inter_body, 0)
                else:
                    perm = jnp.bitwise_xor(iota, j)
                    is_hi = (iota & j) != 0

                    def intra_body(v, _, k=k, j=j, perm=perm, is_hi=is_hi):
                        ak = skey_v[pl.ds(v * _L, _L)]
                        ai = sidx_v[pl.ds(v * _L, _L)]
                        bk = _permute(ak, perm)
                        bi = _permute(ai, perm)
                        if k <= 8:
                            asc = (iota & k) == 0
                        else:
                            asc = jnp.broadcast_to(((v * _L) & k) == 0, (_L,))
                        hold_min = asc != is_hi
                        altb = jnp.logical_or(
                            ak < bk, jnp.logical_and(ak == bk, ai < bi))
                        sel = altb == hold_min
                        skey_v[pl.ds(v * _L, _L)] = jnp.where(sel, ak, bk)
                        sidx_v[pl.ds(v * _L, _L)] = jnp.where(sel, ai, bi)
                        return 0
                    lax.fori_loop(0, nvec, intra_body, 0)
                j //= 2
            k *= 2

        pltpu.sync_copy(sidx_v.at[pl.ds(0, _K)], b_out.at[pl.ds(b * _K, _K)])
        for c in range(_D):
            base = (b * _D + c) * _N

            def gi_body(v, _, base=base):
                gidx_v[pl.ds(v * _L, _L)] = (
                    sidx_v[pl.ds(v * _L, _L)] + base)
                return 0
            lax.fori_loop(0, _K // _L, gi_body, 0)
            pltpu.async_copy(
                pc.at[gidx_v], aout_v.at[pl.ds(c * _K, _K)], sem).wait()
        pltpu.sync_copy(aout_v, a_out.at[pl.ds(b * _D * _K, _D * _K)])


def _sc_topk(dist, pcloud):
    mesh = plsc.VectorSubcoreMesh(core_axis_name="c", subcore_axis_name="s")
    f = pl.kernel(
        _sc_body,
        out_type=(
            jax.ShapeDtypeStruct((_B * _D * _K,), jnp.float32),
            jax.ShapeDtypeStruct((_B * _K,), jnp.int32),
        ),
        mesh=mesh,
        compiler_params=pltpu.CompilerParams(needs_layout_passes=False),
        scratch_types=[
            pltpu.VMEM((_N,), jnp.float32),
            pltpu.VMEM((_L * _BINS,), jnp.int32),
            pltpu.VMEM((_CAND + _L,), jnp.float32),
            pltpu.VMEM((_CAND + _L,), jnp.int32),
            pltpu.VMEM((_CAND + _L,), jnp.float32),
            pltpu.VMEM((_CAND + _L,), jnp.int32),
            pltpu.VMEM((_D * _K,), jnp.float32),
            pltpu.VMEM((_K,), jnp.int32),
            pltpu.SemaphoreType.DMA,
        ],
    )
    a_f, b_f = f(dist.reshape(_B * _N), pcloud.reshape(_B * _D * _N))
    return a_f.reshape(_B, _D, _K), b_f.reshape(_B, _K)


def kernel(pcloud, P1, K):
    dist = _norms(pcloud, P1).reshape(_B, _N)
    a, idx = _sc_topk(dist, pcloud)
    off = (jnp.asarray(K) - _K).astype(jnp.int32)
    return (a, idx + off)
